# baseline (speedup 1.0000x reference)
import functools
import math

import jax
import jax.numpy as jnp
from jax import lax
from jax.experimental import pallas as pl
from jax.experimental.pallas import tpu as pltpu
from jax.experimental.pallas import tpu_sc as plsc

LAMBDA_SUP = 10.0
LAMBDA_CON = 1.0
LAMBDA_AFF = 0.1
LAMBDA_DIST = 0.1
LAMBDA_BDY = 0.5
WARMUP_EPOCHS = 15
IGNORE_INDEX = 255

BLK = 4096


def _dense_a_kernel(sem_ref, geo_ref, tgt_ref, feat_ref, proto_ref,
                    out_ref, acc_ref):
    i = pl.program_id(0)
    nsteps = pl.num_programs(0)
    blk, NCLS = sem_ref.shape

    tgt = tgt_ref[...]
    valid = (tgt != IGNORE_INDEX)
    validf = valid.astype(jnp.float32)
    nvalid = jnp.sum(validf)
    cls_iota = jax.lax.broadcasted_iota(jnp.int32, (blk, NCLS), 1)
    onehot = (cls_iota == tgt).astype(jnp.float32)

    def softmax_parts(x):
        m = jnp.max(x, axis=1, keepdims=True)
        e = jnp.exp(x - m)
        s = jnp.sum(e, axis=1, keepdims=True)
        lse = jnp.log(s) + m
        p = e / s
        return p, lse

    sem = sem_ref[...]
    geo = geo_ref[...]
    p_sem, lse_sem = softmax_parts(sem)
    p_geo, lse_geo = softmax_parts(geo)
    nll_sem = (lse_sem[:, 0] - jnp.sum(sem * onehot, axis=1)) * validf[:, 0]
    nll_geo = (lse_geo[:, 0] - jnp.sum(geo * onehot, axis=1)) * validf[:, 0]
    nll_sum = jnp.sum(nll_sem) + jnp.sum(nll_geo)

    eps = 1e-6
    pse = p_sem + eps
    pge = p_geo + eps
    log_pse = jnp.log(pse)
    log_pge = jnp.log(pge)
    kl_sg = jnp.sum(pge * (log_pge - log_pse))
    kl_gs = jnp.sum(pse * (log_pse - log_pge))

    feat = feat_ref[...]
    fnorm = jnp.maximum(jnp.sqrt(jnp.sum(feat * feat, axis=1, keepdims=True)),
                        1e-12)
    nf = feat / fnorm
    proto = proto_ref[...]
    pnorm = jnp.maximum(jnp.sqrt(jnp.sum(proto * proto, axis=1,
                                         keepdims=True)), 1e-12)
    nproto = proto / pnorm
    sim = jax.lax.dot_general(nf, nproto, (((1,), (1,)), ((), ())),
                              preferred_element_type=jnp.float32)
    tsim = jnp.sum(sim * onehot, axis=1)
    dist_sum = jnp.sum(validf[:, 0] * (1.0 - tsim))

    @pl.when(i == 0)
    def _init():
        for j in range(5):
            acc_ref[j] = 0.0

    acc_ref[0] += nll_sum
    acc_ref[1] += nvalid
    acc_ref[2] += kl_sg
    acc_ref[3] += kl_gs
    acc_ref[4] += dist_sum

    @pl.when(i == nsteps - 1)
    def _store():
        for j in range(5):
            out_ref[0, j] = acc_ref[j]


def _dense_a(sem_logits, geo_logits, target2d, feat_flat, prototypes):
    BN, NCLS = sem_logits.shape
    C = feat_flat.shape[1]
    nsteps = BN // BLK
    return pl.pallas_call(
        _dense_a_kernel,
        grid=(nsteps,),
        in_specs=[
            pl.BlockSpec((BLK, NCLS), lambda i: (i, 0)),
            pl.BlockSpec((BLK, NCLS), lambda i: (i, 0)),
            pl.BlockSpec((BLK, 1), lambda i: (i, 0)),
            pl.BlockSpec((BLK, C), lambda i: (i, 0)),
            pl.BlockSpec((prototypes.shape[0], C), lambda i: (0, 0)),
        ],
        out_specs=pl.BlockSpec(memory_space=pltpu.SMEM),
        out_shape=jax.ShapeDtypeStruct((1, 5), jnp.float32),
        scratch_shapes=[pltpu.SMEM((5,), jnp.float32)],
    )(sem_logits, geo_logits, target2d, feat_flat, prototypes)


def _dense_b_kernel(epoch_ref, parts_ref, aff_ref, d2s_ref, d2i_ref, bdy_ref,
                    out_ref, acc_ref):
    i = pl.program_id(0)
    nsteps = pl.num_programs(0)
    blk, K = aff_ref.shape
    C = 128

    aff = aff_ref[...]
    d2s = d2s_ref[...]
    amask = (aff > 0.8).astype(jnp.float32)
    aff_num = jnp.sum(aff * d2s * amask) * (1.0 / math.sqrt(C))
    mask_sum = jnp.sum(amask)

    d2i = d2i_ref[...]
    jd = jnp.sqrt(d2i)
    es = jnp.sum(jd, axis=1) * (1.0 / K)
    tb = jax.nn.sigmoid((es - 0.15) * 20.0)
    x = bdy_ref[...][:, 0]
    bce = jnp.maximum(x, 0.0) - x * tb + jnp.log1p(jnp.exp(-jnp.abs(x)))
    bce_sum = jnp.sum(bce)

    @pl.when(i == 0)
    def _init():
        for j in range(3):
            acc_ref[j] = 0.0

    acc_ref[0] += aff_num
    acc_ref[1] += mask_sum
    acc_ref[2] += bce_sum

    @pl.when(i == nsteps - 1)
    def _finalize():
        BN = blk * nsteps
        nv = jnp.maximum(parts_ref[0, 1], 1.0)
        loss_sup = parts_ref[0, 0] / nv
        epoch = epoch_ref[0]
        in_warmup = epoch < WARMUP_EPOCHS
        progress = jnp.clip(
            (epoch.astype(jnp.float32) - 1.0) / WARMUP_EPOCHS, 0.0, 1.0)
        lam_con = jnp.where(in_warmup, LAMBDA_CON * progress * 0.1,
                            LAMBDA_CON)
        kl_sg_m = parts_ref[0, 2] / BN
        kl_gs_m = parts_ref[0, 3] / BN
        loss_con = jnp.where(in_warmup, kl_sg_m, (kl_sg_m + kl_gs_m) * 0.5)
        loss_aff = acc_ref[0] / (acc_ref[1] + 1e-6)
        loss_dist = parts_ref[0, 4] / nv
        loss_bdy = acc_ref[2] / BN
        out_ref[0, 0] = (loss_sup * LAMBDA_SUP + loss_con * lam_con
                         + loss_aff * LAMBDA_AFF + loss_dist * LAMBDA_DIST
                         + loss_bdy * LAMBDA_BDY)


def _dense_b(epoch_arr, parts, aff_flat, d2s, d2i, bdy_flat):
    BN, K = aff_flat.shape
    nsteps = BN // BLK
    out = pl.pallas_call(
        _dense_b_kernel,
        grid=(nsteps,),
        in_specs=[
            pl.BlockSpec(memory_space=pltpu.SMEM),
            pl.BlockSpec(memory_space=pltpu.SMEM),
            pl.BlockSpec((BLK, K), lambda i: (i, 0)),
            pl.BlockSpec((BLK, K), lambda i: (i, 0)),
            pl.BlockSpec((BLK, K), lambda i: (i, 0)),
            pl.BlockSpec((BLK, 1), lambda i: (i, 0)),
        ],
        out_specs=pl.BlockSpec(memory_space=pltpu.SMEM),
        out_shape=jax.ShapeDtypeStruct((1, 1), jnp.float32),
        scratch_shapes=[pltpu.SMEM((3,), jnp.float32)],
    )(epoch_arr, parts, aff_flat, d2s, d2i, bdy_flat)
    return out[0, 0]


def _make_sc_dist2(BN, K, C, D, N):
    info = plsc.get_sparse_core_info()
    NC, NS = info.num_cores, info.num_subcores
    NW = NC * NS
    per_w = BN // NW
    CH = 16
    NCHUNK = per_w // CH
    E = CH * K
    CP = C // 2
    DP = D // 2
    mesh = plsc.VectorSubcoreMesh(core_axis_name="c", subcore_axis_name="s")

    @functools.partial(
        pl.kernel,
        mesh=mesh,
        out_type=[
            jax.ShapeDtypeStruct((BN, K), jnp.float32),
            jax.ShapeDtypeStruct((BN, K), jnp.float32),
        ],
        scratch_types=[
            pltpu.VMEM((E,), jnp.int32),
            pltpu.VMEM((E,), jnp.int32),
            pltpu.VMEM((E, CP), jnp.int32),
            pltpu.VMEM((E, CP), jnp.int32),
            pltpu.VMEM((E, DP), jnp.int32),
            pltpu.VMEM((E, DP), jnp.int32),
            pltpu.VMEM((CH, CP), jnp.int32),
            pltpu.VMEM((CH, DP), jnp.int32),
            pltpu.VMEM((CH, K), jnp.float32),
            pltpu.VMEM((CH, K), jnp.float32),
            pltpu.SemaphoreType.DMA,
            pltpu.SemaphoreType.DMA,
        ],
        compiler_params=pltpu.CompilerParams(needs_layout_passes=False,
                                             use_tc_tiling_on_sc=False),
    )
    def sc_kernel(feat_hbm, inp_hbm, kidx_hbm, d2s_hbm, d2i_hbm,
                  idx0_v, idx1_v, nbrf0_v, nbrf1_v, nbri0_v, nbri1_v,
                  cenf_v, ceni_v, outs_v, outi_v, sem0, sem1):
        wid = lax.axis_index("s") * NC + lax.axis_index("c")
        base_row = wid * per_w
        batch_base = (base_row // N) * N
        idx_bufs = (idx0_v, idx1_v)
        nbrf_bufs = (nbrf0_v, nbrf1_v)
        nbri_bufs = (nbri0_v, nbri1_v)
        sems = (sem0, sem1)
        NACC = 4

        def issue_gather(ch, slot):
            row0 = base_row + ch * CH
            idx_v = idx_bufs[slot]
            pltpu.sync_copy(kidx_hbm.at[pl.ds(row0 * K, E)], idx_v)
            for j in range(E // 16):
                sl = pl.ds(j * 16, 16)
                idx_v[sl] = idx_v[sl] + batch_base
            pltpu.async_copy(feat_hbm.at[idx_v], nbrf_bufs[slot], sems[slot])
            pltpu.async_copy(inp_hbm.at[idx_v], nbri_bufs[slot], sems[slot])

        def wait_gather(slot):
            pltpu.make_async_copy(feat_hbm.at[idx_bufs[slot]],
                                  nbrf_bufs[slot], sems[slot]).wait()
            pltpu.make_async_copy(inp_hbm.at[idx_bufs[slot]],
                                  nbri_bufs[slot], sems[slot]).wait()

        def compute_chunk(ch, slot):
            row0 = base_row + ch * CH
            nbrf_v = nbrf_bufs[slot]
            nbri_v = nbri_bufs[slot]
            pltpu.sync_copy(feat_hbm.at[pl.ds(row0, CH), :], cenf_v)
            pltpu.sync_copy(inp_hbm.at[pl.ds(row0, CH), :], ceni_v)
            wait_gather(slot)
            lane = lax.iota(jnp.int32, 16)

            def center_body(i, _):
                ridx = lane + i * K

                def unpack2(x):
                    return plsc.unpack(plsc.bitcast(x, jnp.bfloat16),
                                       format=plsc.PackFormat.INTERLEAVED)

                def dist2_table(nbr_v, cen_v, ncols_p, out_v):
                    accs = [jnp.zeros((16,), jnp.float32)
                            for _ in range(NACC)]
                    for g in range(ncols_p // 16):
                        cpg = cen_v[i, pl.ds(g * 16, 16)]
                        for s in range(16):
                            rot = (lane + s) & 15
                            col = rot + (g * 16)
                            gp = plsc.load_gather(nbr_v, [ridx, col])
                            cpr = jnp.take_along_axis(cpg, rot, axis=0)
                            ge, go = unpack2(gp)
                            ce, co = unpack2(cpr)
                            d0 = ge - ce
                            d1 = go - co
                            accs[s % NACC] = (accs[s % NACC]
                                              + (d0 * d0 + d1 * d1))
                    while len(accs) > 1:
                        accs = [a + b for a, b in zip(accs[::2], accs[1::2])]
                    out_v[i, :] = accs[0]

                dist2_table(nbrf_v, cenf_v, CP, outs_v)
                dist2_table(nbri_v, ceni_v, DP, outi_v)
                return _

            lax.fori_loop(0, CH, center_body, None)
            pltpu.sync_copy(outs_v, d2s_hbm.at[pl.ds(row0, CH), :])
            pltpu.sync_copy(outi_v, d2i_hbm.at[pl.ds(row0, CH), :])

        issue_gather(0, 0)

        def pair_body(h, _):
            ch0 = h * 2
            ch1 = ch0 + 1
            issue_gather(ch1, 1)
            compute_chunk(ch0, 0)

            @pl.when(ch1 + 1 < NCHUNK)
            def _():
                issue_gather(ch1 + 1, 0)

            compute_chunk(ch1, 1)
            return _

        lax.fori_loop(0, NCHUNK // 2, pair_body, None)

    return sc_kernel


def kernel(sem_logits, geo_logits, sem_feat_dense, affinity, prototypes,
           input_jafar_feat, bdy_logits, target, k_idx, epoch):
    B, N, C = sem_feat_dense.shape
    K = k_idx.shape[-1]
    D = input_jafar_feat.shape[-1]
    BN = B * N

    feat_flat = sem_feat_dense.reshape(BN, C)
    inp_flat = input_jafar_feat.reshape(BN, D)
    kidx_flat = k_idx.reshape(BN * K)

    featp = jax.lax.bitcast_convert_type(
        feat_flat.astype(jnp.bfloat16).reshape(BN, C // 2, 2), jnp.int32)
    inpp = jax.lax.bitcast_convert_type(
        inp_flat.astype(jnp.bfloat16).reshape(BN, D // 2, 2), jnp.int32)

    sc_kernel = _make_sc_dist2(BN, K, C, D, N)
    d2s, d2i = sc_kernel(featp, inpp, kidx_flat)

    epoch_arr = jnp.asarray(epoch, dtype=jnp.int32).reshape(1)
    target2d = target.reshape(BN, 1)
    bdy_flat = bdy_logits.reshape(BN, 1)
    aff_flat = affinity.reshape(BN, K)

    parts = _dense_a(sem_logits, geo_logits, target2d, feat_flat, prototypes)
    return _dense_b(epoch_arr, parts, aff_flat, d2s, d2i, bdy_flat)


# speedup vs baseline: 1.3958x; 1.3958x over previous
import functools
import math

import jax
import jax.numpy as jnp
from jax import lax
from jax.experimental import pallas as pl
from jax.experimental.pallas import tpu as pltpu
from jax.experimental.pallas import tpu_sc as plsc

LAMBDA_SUP = 10.0
LAMBDA_CON = 1.0
LAMBDA_AFF = 0.1
LAMBDA_DIST = 0.1
LAMBDA_BDY = 0.5
WARMUP_EPOCHS = 15
IGNORE_INDEX = 255

BLK = 4096


def _dense_a_kernel(sem_ref, geo_ref, tgt_ref, feat_ref, proto_ref,
                    out_ref, acc_ref):
    i = pl.program_id(0)
    nsteps = pl.num_programs(0)
    blk, NCLS = sem_ref.shape

    tgt = tgt_ref[...]
    valid = (tgt != IGNORE_INDEX)
    validf = valid.astype(jnp.float32)
    nvalid = jnp.sum(validf)
    cls_iota = jax.lax.broadcasted_iota(jnp.int32, (blk, NCLS), 1)
    onehot = (cls_iota == tgt).astype(jnp.float32)

    def softmax_parts(x):
        m = jnp.max(x, axis=1, keepdims=True)
        e = jnp.exp(x - m)
        s = jnp.sum(e, axis=1, keepdims=True)
        lse = jnp.log(s) + m
        p = e / s
        return p, lse

    sem = sem_ref[...]
    geo = geo_ref[...]
    p_sem, lse_sem = softmax_parts(sem)
    p_geo, lse_geo = softmax_parts(geo)
    nll_sem = (lse_sem[:, 0] - jnp.sum(sem * onehot, axis=1)) * validf[:, 0]
    nll_geo = (lse_geo[:, 0] - jnp.sum(geo * onehot, axis=1)) * validf[:, 0]
    nll_sum = jnp.sum(nll_sem) + jnp.sum(nll_geo)

    eps = 1e-6
    pse = p_sem + eps
    pge = p_geo + eps
    log_pse = jnp.log(pse)
    log_pge = jnp.log(pge)
    kl_sg = jnp.sum(pge * (log_pge - log_pse))
    kl_gs = jnp.sum(pse * (log_pse - log_pge))

    feat = feat_ref[...]
    fnorm = jnp.maximum(jnp.sqrt(jnp.sum(feat * feat, axis=1, keepdims=True)),
                        1e-12)
    nf = feat / fnorm
    proto = proto_ref[...]
    pnorm = jnp.maximum(jnp.sqrt(jnp.sum(proto * proto, axis=1,
                                         keepdims=True)), 1e-12)
    nproto = proto / pnorm
    sim = jax.lax.dot_general(nf, nproto, (((1,), (1,)), ((), ())),
                              preferred_element_type=jnp.float32)
    tsim = jnp.sum(sim * onehot, axis=1)
    dist_sum = jnp.sum(validf[:, 0] * (1.0 - tsim))

    @pl.when(i == 0)
    def _init():
        for j in range(5):
            acc_ref[j] = 0.0

    acc_ref[0] += nll_sum
    acc_ref[1] += nvalid
    acc_ref[2] += kl_sg
    acc_ref[3] += kl_gs
    acc_ref[4] += dist_sum

    @pl.when(i == nsteps - 1)
    def _store():
        for j in range(5):
            out_ref[0, j] = acc_ref[j]


def _dense_a(sem_logits, geo_logits, target2d, feat_flat, prototypes):
    BN, NCLS = sem_logits.shape
    C = feat_flat.shape[1]
    nsteps = BN // BLK
    return pl.pallas_call(
        _dense_a_kernel,
        grid=(nsteps,),
        in_specs=[
            pl.BlockSpec((BLK, NCLS), lambda i: (i, 0)),
            pl.BlockSpec((BLK, NCLS), lambda i: (i, 0)),
            pl.BlockSpec((BLK, 1), lambda i: (i, 0)),
            pl.BlockSpec((BLK, C), lambda i: (i, 0)),
            pl.BlockSpec((prototypes.shape[0], C), lambda i: (0, 0)),
        ],
        out_specs=pl.BlockSpec(memory_space=pltpu.SMEM),
        out_shape=jax.ShapeDtypeStruct((1, 5), jnp.float32),
        scratch_shapes=[pltpu.SMEM((5,), jnp.float32)],
    )(sem_logits, geo_logits, target2d, feat_flat, prototypes)


def _dense_b_kernel(epoch_ref, parts_ref, aff_ref, d2s_ref, d2i_ref, bdy_ref,
                    out_ref, acc_ref):
    i = pl.program_id(0)
    nsteps = pl.num_programs(0)
    blk, K = aff_ref.shape
    C = 128

    aff = aff_ref[...]
    d2s = d2s_ref[...]
    amask = (aff > 0.8).astype(jnp.float32)
    aff_num = jnp.sum(aff * d2s * amask) * (1.0 / math.sqrt(C))
    mask_sum = jnp.sum(amask)

    d2i = d2i_ref[...]
    jd = jnp.sqrt(d2i)
    es = jnp.sum(jd, axis=1) * (1.0 / K)
    tb = jax.nn.sigmoid((es - 0.15) * 20.0)
    x = bdy_ref[...][:, 0]
    bce = jnp.maximum(x, 0.0) - x * tb + jnp.log1p(jnp.exp(-jnp.abs(x)))
    bce_sum = jnp.sum(bce)

    @pl.when(i == 0)
    def _init():
        for j in range(3):
            acc_ref[j] = 0.0

    acc_ref[0] += aff_num
    acc_ref[1] += mask_sum
    acc_ref[2] += bce_sum

    @pl.when(i == nsteps - 1)
    def _finalize():
        BN = blk * nsteps
        nv = jnp.maximum(parts_ref[0, 1], 1.0)
        loss_sup = parts_ref[0, 0] / nv
        epoch = epoch_ref[0]
        in_warmup = epoch < WARMUP_EPOCHS
        progress = jnp.clip(
            (epoch.astype(jnp.float32) - 1.0) / WARMUP_EPOCHS, 0.0, 1.0)
        lam_con = jnp.where(in_warmup, LAMBDA_CON * progress * 0.1,
                            LAMBDA_CON)
        kl_sg_m = parts_ref[0, 2] / BN
        kl_gs_m = parts_ref[0, 3] / BN
        loss_con = jnp.where(in_warmup, kl_sg_m, (kl_sg_m + kl_gs_m) * 0.5)
        loss_aff = acc_ref[0] / (acc_ref[1] + 1e-6)
        loss_dist = parts_ref[0, 4] / nv
        loss_bdy = acc_ref[2] / BN
        out_ref[0, 0] = (loss_sup * LAMBDA_SUP + loss_con * lam_con
                         + loss_aff * LAMBDA_AFF + loss_dist * LAMBDA_DIST
                         + loss_bdy * LAMBDA_BDY)


def _dense_b(epoch_arr, parts, aff_flat, d2s, d2i, bdy_flat):
    BN, K = aff_flat.shape
    nsteps = BN // BLK
    out = pl.pallas_call(
        _dense_b_kernel,
        grid=(nsteps,),
        in_specs=[
            pl.BlockSpec(memory_space=pltpu.SMEM),
            pl.BlockSpec(memory_space=pltpu.SMEM),
            pl.BlockSpec((BLK, K), lambda i: (i, 0)),
            pl.BlockSpec((BLK, K), lambda i: (i, 0)),
            pl.BlockSpec((BLK, K), lambda i: (i, 0)),
            pl.BlockSpec((BLK, 1), lambda i: (i, 0)),
        ],
        out_specs=pl.BlockSpec(memory_space=pltpu.SMEM),
        out_shape=jax.ShapeDtypeStruct((1, 1), jnp.float32),
        scratch_shapes=[pltpu.SMEM((3,), jnp.float32)],
    )(epoch_arr, parts, aff_flat, d2s, d2i, bdy_flat)
    return out[0, 0]


def _make_sc_dist2(BN, K, C, D, N):
    info = plsc.get_sparse_core_info()
    NC, NS = info.num_cores, info.num_subcores
    NW = NC * NS
    per_w = BN // NW
    CH = 16
    NCHUNK = per_w // CH
    E = CH * K
    mesh = plsc.VectorSubcoreMesh(core_axis_name="c", subcore_axis_name="s")

    @functools.partial(
        pl.kernel,
        mesh=mesh,
        out_type=[
            jax.ShapeDtypeStruct((BN, K), jnp.float32),
            jax.ShapeDtypeStruct((BN, K), jnp.float32),
        ],
        scratch_types=[
            pltpu.VMEM((E,), jnp.int32),
            pltpu.VMEM((E,), jnp.int32),
            pltpu.VMEM((E, C), jnp.float32),
            pltpu.VMEM((E, C), jnp.float32),
            pltpu.VMEM((E, D), jnp.float32),
            pltpu.VMEM((E, D), jnp.float32),
            pltpu.VMEM((CH, C), jnp.float32),
            pltpu.VMEM((CH, C), jnp.float32),
            pltpu.VMEM((CH, D), jnp.float32),
            pltpu.VMEM((CH, D), jnp.float32),
            pltpu.VMEM((CH, K), jnp.float32),
            pltpu.VMEM((CH, K), jnp.float32),
            pltpu.SemaphoreType.DMA,
            pltpu.SemaphoreType.DMA,
        ],
        compiler_params=pltpu.CompilerParams(needs_layout_passes=False,
                                             use_tc_tiling_on_sc=False),
    )
    def sc_kernel(feat_hbm, inp_hbm, kidx_hbm, d2s_hbm, d2i_hbm,
                  idx0_v, idx1_v, nbrf0_v, nbrf1_v, nbri0_v, nbri1_v,
                  cenf0_v, cenf1_v, ceni0_v, ceni1_v, outs_v, outi_v,
                  sem0, sem1):
        wid = lax.axis_index("s") * NC + lax.axis_index("c")
        base_row = wid * per_w
        batch_base = (base_row // N) * N
        idx_bufs = (idx0_v, idx1_v)
        nbrf_bufs = (nbrf0_v, nbrf1_v)
        nbri_bufs = (nbri0_v, nbri1_v)
        cenf_bufs = (cenf0_v, cenf1_v)
        ceni_bufs = (ceni0_v, ceni1_v)
        sems = (sem0, sem1)
        NACC = 4

        def issue_gather(ch, slot):
            row0 = base_row + ch * CH
            idx_v = idx_bufs[slot]
            pltpu.sync_copy(kidx_hbm.at[pl.ds(row0 * K, E)], idx_v)
            for j in range(E // 16):
                sl = pl.ds(j * 16, 16)
                idx_v[sl] = idx_v[sl] + batch_base
            pltpu.async_copy(feat_hbm.at[idx_v], nbrf_bufs[slot], sems[slot])
            pltpu.async_copy(inp_hbm.at[idx_v], nbri_bufs[slot], sems[slot])
            pltpu.async_copy(feat_hbm.at[pl.ds(row0, CH), :],
                             cenf_bufs[slot], sems[slot])
            pltpu.async_copy(inp_hbm.at[pl.ds(row0, CH), :],
                             ceni_bufs[slot], sems[slot])

        def wait_gather(ch, slot):
            row0 = base_row + ch * CH
            pltpu.make_async_copy(feat_hbm.at[idx_bufs[slot]],
                                  nbrf_bufs[slot], sems[slot]).wait()
            pltpu.make_async_copy(inp_hbm.at[idx_bufs[slot]],
                                  nbri_bufs[slot], sems[slot]).wait()
            pltpu.make_async_copy(feat_hbm.at[pl.ds(row0, CH), :],
                                  cenf_bufs[slot], sems[slot]).wait()
            pltpu.make_async_copy(inp_hbm.at[pl.ds(row0, CH), :],
                                  ceni_bufs[slot], sems[slot]).wait()

        def compute_chunk(ch, slot):
            row0 = base_row + ch * CH
            nbrf_v = nbrf_bufs[slot]
            nbri_v = nbri_bufs[slot]
            cenf_v = cenf_bufs[slot]
            ceni_v = ceni_bufs[slot]
            wait_gather(ch, slot)
            lane = lax.iota(jnp.int32, 16)

            def center_body(i, _):
                ridx = lane + i * K

                def dist2_table(nbr_v, cen_v, ncols, out_v):
                    accs = [jnp.zeros((16,), jnp.float32)
                            for _ in range(NACC)]
                    for g in range(ncols // 16):
                        cfg = cen_v[i, pl.ds(g * 16, 16)]
                        for s in range(16):
                            rot = (lane + s) & 15
                            col = rot + (g * 16)
                            dv = (plsc.load_gather(nbr_v, [ridx, col])
                                  - jnp.take_along_axis(cfg, rot, axis=0))
                            accs[s % NACC] = accs[s % NACC] + dv * dv
                    while len(accs) > 1:
                        accs = [a + b for a, b in zip(accs[::2], accs[1::2])]
                    out_v[i, :] = accs[0]

                dist2_table(nbrf_v, cenf_v, C, outs_v)
                dist2_table(nbri_v, ceni_v, D, outi_v)
                return _

            lax.fori_loop(0, CH, center_body, None)
            pltpu.sync_copy(outs_v, d2s_hbm.at[pl.ds(row0, CH), :])
            pltpu.sync_copy(outi_v, d2i_hbm.at[pl.ds(row0, CH), :])

        issue_gather(0, 0)

        def pair_body(h, _):
            ch0 = h * 2
            ch1 = ch0 + 1
            issue_gather(ch1, 1)
            compute_chunk(ch0, 0)

            @pl.when(ch1 + 1 < NCHUNK)
            def _():
                issue_gather(ch1 + 1, 0)

            compute_chunk(ch1, 1)
            return _

        lax.fori_loop(0, NCHUNK // 2, pair_body, None)

    return sc_kernel


def kernel(sem_logits, geo_logits, sem_feat_dense, affinity, prototypes,
           input_jafar_feat, bdy_logits, target, k_idx, epoch):
    B, N, C = sem_feat_dense.shape
    K = k_idx.shape[-1]
    D = input_jafar_feat.shape[-1]
    BN = B * N

    feat_flat = sem_feat_dense.reshape(BN, C)
    inp_flat = input_jafar_feat.reshape(BN, D)
    kidx_flat = k_idx.reshape(BN * K)

    sc_kernel = _make_sc_dist2(BN, K, C, D, N)
    d2s, d2i = sc_kernel(feat_flat, inp_flat, kidx_flat)

    epoch_arr = jnp.asarray(epoch, dtype=jnp.int32).reshape(1)
    target2d = target.reshape(BN, 1)
    bdy_flat = bdy_logits.reshape(BN, 1)
    aff_flat = affinity.reshape(BN, K)

    parts = _dense_a(sem_logits, geo_logits, target2d, feat_flat, prototypes)
    return _dense_b(epoch_arr, parts, aff_flat, d2s, d2i, bdy_flat)


# speedup vs baseline: 1.6158x; 1.1576x over previous
import functools
import math

import jax
import jax.numpy as jnp
from jax import lax
from jax.experimental import pallas as pl
from jax.experimental.pallas import tpu as pltpu
from jax.experimental.pallas import tpu_sc as plsc

LAMBDA_SUP = 10.0
LAMBDA_CON = 1.0
LAMBDA_AFF = 0.1
LAMBDA_DIST = 0.1
LAMBDA_BDY = 0.5
WARMUP_EPOCHS = 15
IGNORE_INDEX = 255

BLK = 4096


def _dense_a_kernel(sem_ref, geo_ref, tgt_ref, feat_ref, proto_ref,
                    out_ref, acc_ref):
    i = pl.program_id(0)
    nsteps = pl.num_programs(0)
    blk, NCLS = sem_ref.shape

    tgt = tgt_ref[...]
    valid = (tgt != IGNORE_INDEX)
    validf = valid.astype(jnp.float32)
    nvalid = jnp.sum(validf)
    cls_iota = jax.lax.broadcasted_iota(jnp.int32, (blk, NCLS), 1)
    onehot = (cls_iota == tgt).astype(jnp.float32)

    def softmax_parts(x):
        m = jnp.max(x, axis=1, keepdims=True)
        e = jnp.exp(x - m)
        s = jnp.sum(e, axis=1, keepdims=True)
        lse = jnp.log(s) + m
        p = e / s
        return p, lse

    sem = sem_ref[...]
    geo = geo_ref[...]
    p_sem, lse_sem = softmax_parts(sem)
    p_geo, lse_geo = softmax_parts(geo)
    nll_sem = (lse_sem[:, 0] - jnp.sum(sem * onehot, axis=1)) * validf[:, 0]
    nll_geo = (lse_geo[:, 0] - jnp.sum(geo * onehot, axis=1)) * validf[:, 0]
    nll_sum = jnp.sum(nll_sem) + jnp.sum(nll_geo)

    eps = 1e-6
    pse = p_sem + eps
    pge = p_geo + eps
    log_pse = jnp.log(pse)
    log_pge = jnp.log(pge)
    kl_sg = jnp.sum(pge * (log_pge - log_pse))
    kl_gs = jnp.sum(pse * (log_pse - log_pge))

    feat = feat_ref[...]
    fnorm = jnp.maximum(jnp.sqrt(jnp.sum(feat * feat, axis=1, keepdims=True)),
                        1e-12)
    nf = feat / fnorm
    proto = proto_ref[...]
    pnorm = jnp.maximum(jnp.sqrt(jnp.sum(proto * proto, axis=1,
                                         keepdims=True)), 1e-12)
    nproto = proto / pnorm
    sim = jax.lax.dot_general(nf, nproto, (((1,), (1,)), ((), ())),
                              preferred_element_type=jnp.float32)
    tsim = jnp.sum(sim * onehot, axis=1)
    dist_sum = jnp.sum(validf[:, 0] * (1.0 - tsim))

    @pl.when(i == 0)
    def _init():
        for j in range(5):
            acc_ref[j] = 0.0

    acc_ref[0] += nll_sum
    acc_ref[1] += nvalid
    acc_ref[2] += kl_sg
    acc_ref[3] += kl_gs
    acc_ref[4] += dist_sum

    @pl.when(i == nsteps - 1)
    def _store():
        for j in range(5):
            out_ref[0, j] = acc_ref[j]


def _dense_a(sem_logits, geo_logits, target2d, feat_flat, prototypes):
    BN, NCLS = sem_logits.shape
    C = feat_flat.shape[1]
    nsteps = BN // BLK
    return pl.pallas_call(
        _dense_a_kernel,
        grid=(nsteps,),
        in_specs=[
            pl.BlockSpec((BLK, NCLS), lambda i: (i, 0)),
            pl.BlockSpec((BLK, NCLS), lambda i: (i, 0)),
            pl.BlockSpec((BLK, 1), lambda i: (i, 0)),
            pl.BlockSpec((BLK, C), lambda i: (i, 0)),
            pl.BlockSpec((prototypes.shape[0], C), lambda i: (0, 0)),
        ],
        out_specs=pl.BlockSpec(memory_space=pltpu.SMEM),
        out_shape=jax.ShapeDtypeStruct((1, 5), jnp.float32),
        scratch_shapes=[pltpu.SMEM((5,), jnp.float32)],
    )(sem_logits, geo_logits, target2d, feat_flat, prototypes)


def _dense_b_kernel(epoch_ref, parts_ref, aff_ref, d2s_ref, d2i_ref, bdy_ref,
                    out_ref, acc_ref):
    i = pl.program_id(0)
    nsteps = pl.num_programs(0)
    blk, K = aff_ref.shape
    C = 128

    aff = aff_ref[...]
    d2s = d2s_ref[...]
    amask = (aff > 0.8).astype(jnp.float32)
    aff_num = jnp.sum(aff * d2s * amask) * (1.0 / math.sqrt(C))
    mask_sum = jnp.sum(amask)

    d2i = d2i_ref[...]
    jd = jnp.sqrt(d2i)
    es = jnp.sum(jd, axis=1) * (1.0 / K)
    tb = jax.nn.sigmoid((es - 0.15) * 20.0)
    x = bdy_ref[...][:, 0]
    bce = jnp.maximum(x, 0.0) - x * tb + jnp.log1p(jnp.exp(-jnp.abs(x)))
    bce_sum = jnp.sum(bce)

    @pl.when(i == 0)
    def _init():
        for j in range(3):
            acc_ref[j] = 0.0

    acc_ref[0] += aff_num
    acc_ref[1] += mask_sum
    acc_ref[2] += bce_sum

    @pl.when(i == nsteps - 1)
    def _finalize():
        BN = blk * nsteps
        nv = jnp.maximum(parts_ref[0, 1], 1.0)
        loss_sup = parts_ref[0, 0] / nv
        epoch = epoch_ref[0]
        in_warmup = epoch < WARMUP_EPOCHS
        progress = jnp.clip(
            (epoch.astype(jnp.float32) - 1.0) / WARMUP_EPOCHS, 0.0, 1.0)
        lam_con = jnp.where(in_warmup, LAMBDA_CON * progress * 0.1,
                            LAMBDA_CON)
        kl_sg_m = parts_ref[0, 2] / BN
        kl_gs_m = parts_ref[0, 3] / BN
        loss_con = jnp.where(in_warmup, kl_sg_m, (kl_sg_m + kl_gs_m) * 0.5)
        loss_aff = acc_ref[0] / (acc_ref[1] + 1e-6)
        loss_dist = parts_ref[0, 4] / nv
        loss_bdy = acc_ref[2] / BN
        out_ref[0, 0] = (loss_sup * LAMBDA_SUP + loss_con * lam_con
                         + loss_aff * LAMBDA_AFF + loss_dist * LAMBDA_DIST
                         + loss_bdy * LAMBDA_BDY)


def _dense_b(epoch_arr, parts, aff_flat, d2s, d2i, bdy_flat):
    BN, K = aff_flat.shape
    nsteps = BN // BLK
    out = pl.pallas_call(
        _dense_b_kernel,
        grid=(nsteps,),
        in_specs=[
            pl.BlockSpec(memory_space=pltpu.SMEM),
            pl.BlockSpec(memory_space=pltpu.SMEM),
            pl.BlockSpec((BLK, K), lambda i: (i, 0)),
            pl.BlockSpec((BLK, K), lambda i: (i, 0)),
            pl.BlockSpec((BLK, K), lambda i: (i, 0)),
            pl.BlockSpec((BLK, 1), lambda i: (i, 0)),
        ],
        out_specs=pl.BlockSpec(memory_space=pltpu.SMEM),
        out_shape=jax.ShapeDtypeStruct((1, 1), jnp.float32),
        scratch_shapes=[pltpu.SMEM((3,), jnp.float32)],
    )(epoch_arr, parts, aff_flat, d2s, d2i, bdy_flat)
    return out[0, 0]


def _make_sc_dist2(BN, K, C, D, N):
    info = plsc.get_sparse_core_info()
    NC, NS = info.num_cores, info.num_subcores
    NW = NC * NS
    per_w = BN // NW
    CH = 16
    NCHUNK = per_w // CH
    E = CH * K
    mesh = plsc.VectorSubcoreMesh(core_axis_name="c", subcore_axis_name="s")

    @functools.partial(
        pl.kernel,
        mesh=mesh,
        out_type=[
            jax.ShapeDtypeStruct((BN, K), jnp.float32),
            jax.ShapeDtypeStruct((BN, K), jnp.float32),
        ],
        scratch_types=[
            pltpu.VMEM((E,), jnp.int32),
            pltpu.VMEM((E,), jnp.int32),
            pltpu.VMEM((E, C), jnp.float32),
            pltpu.VMEM((E, C), jnp.float32),
            pltpu.VMEM((E, D), jnp.float32),
            pltpu.VMEM((E, D), jnp.float32),
            pltpu.VMEM((CH, C), jnp.float32),
            pltpu.VMEM((CH, C), jnp.float32),
            pltpu.VMEM((CH, D), jnp.float32),
            pltpu.VMEM((CH, D), jnp.float32),
            pltpu.VMEM((CH, K), jnp.float32),
            pltpu.VMEM((CH, K), jnp.float32),
            pltpu.VMEM((CH, K), jnp.float32),
            pltpu.VMEM((CH, K), jnp.float32),
            pltpu.SemaphoreType.DMA,
            pltpu.SemaphoreType.DMA,
            pltpu.SemaphoreType.DMA,
            pltpu.SemaphoreType.DMA,
            pltpu.SemaphoreType.DMA,
            pltpu.SemaphoreType.DMA,
        ],
        compiler_params=pltpu.CompilerParams(needs_layout_passes=False,
                                             use_tc_tiling_on_sc=False),
    )
    def sc_kernel(feat_hbm, inp_hbm, kidx_hbm, d2s_hbm, d2i_hbm,
                  idx0_v, idx1_v, nbrf0_v, nbrf1_v, nbri0_v, nbri1_v,
                  cenf0_v, cenf1_v, ceni0_v, ceni1_v,
                  outs0_v, outs1_v, outi0_v, outi1_v,
                  sem0, sem1, isem0, isem1, osem0, osem1):
        wid = lax.axis_index("s") * NC + lax.axis_index("c")
        base_row = wid * per_w
        batch_base = (base_row // N) * N
        idx_bufs = (idx0_v, idx1_v)
        nbrf_bufs = (nbrf0_v, nbrf1_v)
        nbri_bufs = (nbri0_v, nbri1_v)
        cenf_bufs = (cenf0_v, cenf1_v)
        ceni_bufs = (ceni0_v, ceni1_v)
        outs_bufs = (outs0_v, outs1_v)
        outi_bufs = (outi0_v, outi1_v)
        sems = (sem0, sem1)
        isems = (isem0, isem1)
        osems = (osem0, osem1)
        NACC = 4

        def idx_start(ch, slot):
            row0 = base_row + ch * CH
            pltpu.async_copy(kidx_hbm.at[pl.ds(row0 * K, E)],
                             idx_bufs[slot], isems[slot])

        def fire(ch, slot):
            row0 = base_row + ch * CH
            idx_v = idx_bufs[slot]
            pltpu.make_async_copy(kidx_hbm.at[pl.ds(row0 * K, E)],
                                  idx_v, isems[slot]).wait()
            for j in range(E // 16):
                sl = pl.ds(j * 16, 16)
                idx_v[sl] = idx_v[sl] + batch_base
            pltpu.async_copy(feat_hbm.at[idx_v], nbrf_bufs[slot], sems[slot])
            pltpu.async_copy(inp_hbm.at[idx_v], nbri_bufs[slot], sems[slot])
            pltpu.async_copy(feat_hbm.at[pl.ds(row0, CH), :],
                             cenf_bufs[slot], sems[slot])
            pltpu.async_copy(inp_hbm.at[pl.ds(row0, CH), :],
                             ceni_bufs[slot], sems[slot])

        def wait_gather(ch, slot):
            row0 = base_row + ch * CH
            pltpu.make_async_copy(feat_hbm.at[idx_bufs[slot]],
                                  nbrf_bufs[slot], sems[slot]).wait()
            pltpu.make_async_copy(inp_hbm.at[idx_bufs[slot]],
                                  nbri_bufs[slot], sems[slot]).wait()
            pltpu.make_async_copy(feat_hbm.at[pl.ds(row0, CH), :],
                                  cenf_bufs[slot], sems[slot]).wait()
            pltpu.make_async_copy(inp_hbm.at[pl.ds(row0, CH), :],
                                  ceni_bufs[slot], sems[slot]).wait()

        def compute_chunk(ch, slot, prefetch):
            row0 = base_row + ch * CH
            nbrf_v = nbrf_bufs[slot]
            nbri_v = nbri_bufs[slot]
            cenf_v = cenf_bufs[slot]
            ceni_v = ceni_bufs[slot]
            outs_v = outs_bufs[slot]
            outi_v = outi_bufs[slot]
            wait_gather(ch, slot)
            prefetch()

            @pl.when(ch >= 2)
            def _drain_prev_out():
                row_p = base_row + (ch - 2) * CH
                pltpu.make_async_copy(outs_v, d2s_hbm.at[pl.ds(row_p, CH), :],
                                      osems[slot]).wait()
                pltpu.make_async_copy(outi_v, d2i_hbm.at[pl.ds(row_p, CH), :],
                                      osems[slot]).wait()

            lane = lax.iota(jnp.int32, 16)

            def center_body(i, _):
                ridx = lane + i * K

                def dist2_table(nbr_v, cen_v, ncols, out_v):
                    accs = [jnp.zeros((16,), jnp.float32)
                            for _ in range(NACC)]
                    for g in range(ncols // 16):
                        cfg = cen_v[i, pl.ds(g * 16, 16)]
                        for s in range(16):
                            rot = (lane + s) & 15
                            col = rot + (g * 16)
                            dv = (plsc.load_gather(nbr_v, [ridx, col])
                                  - jnp.take_along_axis(cfg, rot, axis=0))
                            accs[s % NACC] = accs[s % NACC] + dv * dv
                    while len(accs) > 1:
                        accs = [a + b for a, b in zip(accs[::2], accs[1::2])]
                    out_v[i, :] = accs[0]

                dist2_table(nbrf_v, cenf_v, C, outs_v)
                dist2_table(nbri_v, ceni_v, D, outi_v)
                return _

            lax.fori_loop(0, CH, center_body, None)
            pltpu.async_copy(outs_v, d2s_hbm.at[pl.ds(row0, CH), :],
                             osems[slot])
            pltpu.async_copy(outi_v, d2i_hbm.at[pl.ds(row0, CH), :],
                             osems[slot])

        idx_start(0, 0)
        idx_start(1, 1)
        fire(0, 0)
        fire(1, 1)

        def pair_body(h, _):
            ch0 = h * 2
            ch1 = ch0 + 1

            def pf0():
                @pl.when(ch0 + 2 < NCHUNK)
                def _():
                    idx_start(ch0 + 2, 0)

            compute_chunk(ch0, 0, pf0)

            @pl.when(ch0 + 2 < NCHUNK)
            def _():
                fire(ch0 + 2, 0)

            def pf1():
                @pl.when(ch1 + 2 < NCHUNK)
                def _():
                    idx_start(ch1 + 2, 1)

            compute_chunk(ch1, 1, pf1)

            @pl.when(ch1 + 2 < NCHUNK)
            def _():
                fire(ch1 + 2, 1)

            return _

        lax.fori_loop(0, NCHUNK // 2, pair_body, None)

        for slot in (0, 1):
            row_l = base_row + (NCHUNK - 2 + slot) * CH
            pltpu.make_async_copy(outs_bufs[slot],
                                  d2s_hbm.at[pl.ds(row_l, CH), :],
                                  osems[slot]).wait()
            pltpu.make_async_copy(outi_bufs[slot],
                                  d2i_hbm.at[pl.ds(row_l, CH), :],
                                  osems[slot]).wait()

    return sc_kernel


def kernel(sem_logits, geo_logits, sem_feat_dense, affinity, prototypes,
           input_jafar_feat, bdy_logits, target, k_idx, epoch):
    B, N, C = sem_feat_dense.shape
    K = k_idx.shape[-1]
    D = input_jafar_feat.shape[-1]
    BN = B * N

    feat_flat = sem_feat_dense.reshape(BN, C)
    inp_flat = input_jafar_feat.reshape(BN, D)
    kidx_flat = k_idx.reshape(BN * K)

    sc_kernel = _make_sc_dist2(BN, K, C, D, N)
    d2s, d2i = sc_kernel(feat_flat, inp_flat, kidx_flat)

    epoch_arr = jnp.asarray(epoch, dtype=jnp.int32).reshape(1)
    target2d = target.reshape(BN, 1)
    bdy_flat = bdy_logits.reshape(BN, 1)
    aff_flat = affinity.reshape(BN, K)

    parts = _dense_a(sem_logits, geo_logits, target2d, feat_flat, prototypes)
    return _dense_b(epoch_arr, parts, aff_flat, d2s, d2i, bdy_flat)
